# initial kernel scaffold (unmeasured)
import jax
import jax.numpy as jnp
from jax import lax
from jax.experimental import pallas as pl
from jax.experimental.pallas import tpu as pltpu

N_DEV = 4


def _ag_body(x_ref, out_ref, local_sem, send_sems, recv_sems):
    my = lax.axis_index("i")
    left = lax.rem(my + (N_DEV - 1), N_DEV)
    right = lax.rem(my + 1, N_DEV)

    barrier = pltpu.get_barrier_semaphore()
    pl.semaphore_signal(barrier, inc=1, device_id=(left,),
                        device_id_type=pl.DeviceIdType.MESH)
    pl.semaphore_signal(barrier, inc=1, device_id=(right,),
                        device_id_type=pl.DeviceIdType.MESH)
    pl.semaphore_wait(barrier, 2)

    cp = pltpu.make_async_copy(x_ref, out_ref.at[my], local_sem)
    cp.start()
    cp.wait()

    for h in range(N_DEV - 1):
        origin = lax.rem(my - h + N_DEV, N_DEV)
        rdma = pltpu.make_async_remote_copy(
            src_ref=out_ref.at[origin],
            dst_ref=out_ref.at[origin],
            send_sem=send_sems.at[h],
            recv_sem=recv_sems.at[h],
            device_id=(right,),
            device_id_type=pl.DeviceIdType.MESH,
        )
        rdma.start()
        rdma.wait()


def kernel(x, w_mat):
    m_per, k = x.shape

    gathered = pl.pallas_call(
        _ag_body,
        out_shape=jax.ShapeDtypeStruct((N_DEV, m_per, k), x.dtype),
        in_specs=[pl.BlockSpec(memory_space=pltpu.ANY)],
        out_specs=pl.BlockSpec(memory_space=pltpu.ANY),
        scratch_shapes=[
            pltpu.SemaphoreType.DMA,
            pltpu.SemaphoreType.DMA((N_DEV - 1,)),
            pltpu.SemaphoreType.DMA((N_DEV - 1,)),
        ],
        compiler_params=pltpu.CompilerParams(collective_id=0),
    )(x)

    x_full = gathered.reshape(N_DEV * m_per, k)
    y = jnp.dot(x_full, w_mat, preferred_element_type=jnp.float32)
    return y * jax.nn.sigmoid(y)


# baseline (device time: 4406142 ns/iter reference)
import jax
import jax.numpy as jnp
from jax import lax
from jax.experimental import pallas as pl
from jax.experimental.pallas import tpu as pltpu

N_DEV = 4


def _ag_body(x_ref, out_ref, local_sem, send_sems, recv_sems):
    my = lax.axis_index("i")
    left = lax.rem(my + (N_DEV - 1), N_DEV)
    right = lax.rem(my + 1, N_DEV)

    barrier = pltpu.get_barrier_semaphore()
    pl.semaphore_signal(barrier, inc=1, device_id=(left,),
                        device_id_type=pl.DeviceIdType.MESH)
    pl.semaphore_signal(barrier, inc=1, device_id=(right,),
                        device_id_type=pl.DeviceIdType.MESH)
    pl.semaphore_wait(barrier, 2)

    cp = pltpu.make_async_copy(x_ref, out_ref.at[my], local_sem)
    cp.start()
    cp.wait()

    for h in range(N_DEV - 1):
        origin = lax.rem(my - h + N_DEV, N_DEV)
        rdma = pltpu.make_async_remote_copy(
            src_ref=out_ref.at[origin],
            dst_ref=out_ref.at[origin],
            send_sem=send_sems.at[h],
            recv_sem=recv_sems.at[h],
            device_id=(right,),
            device_id_type=pl.DeviceIdType.MESH,
        )
        rdma.start()
        rdma.wait()


def kernel(x, w_mat):
    m_per, k = x.shape

    gathered = pl.pallas_call(
        _ag_body,
        out_shape=jax.ShapeDtypeStruct((N_DEV, m_per, k), x.dtype),
        in_specs=[pl.BlockSpec(memory_space=pltpu.MemorySpace.HBM)],
        out_specs=pl.BlockSpec(memory_space=pltpu.MemorySpace.HBM),
        scratch_shapes=[
            pltpu.SemaphoreType.DMA,
            pltpu.SemaphoreType.DMA((N_DEV - 1,)),
            pltpu.SemaphoreType.DMA((N_DEV - 1,)),
        ],
        compiler_params=pltpu.CompilerParams(collective_id=0),
    )(x)

    x_full = gathered.reshape(N_DEV * m_per, k)
    y = jnp.dot(x_full, w_mat, preferred_element_type=jnp.float32)
    return y * jax.nn.sigmoid(y)


# device time: 3324233 ns/iter; 1.3255x vs baseline; 1.3255x over previous
import jax
import jax.numpy as jnp
from jax import lax
from jax.experimental import pallas as pl
from jax.experimental.pallas import tpu as pltpu

N_DEV = 4


def _ag_body(x_ref, out_ref, local_sem,
             send_r0, recv_r0, send_l0, recv_l0,
             send_r1, recv_r1, send_l1, recv_l1):
    m_per = x_ref.shape[0]
    half = m_per // 2
    my = lax.axis_index("i")
    left = lax.rem(my + (N_DEV - 1), N_DEV)
    right = lax.rem(my + 1, N_DEV)
    lm1 = left
    rp1 = right
    diag = lax.rem(my + 2, N_DEV)

    barrier = pltpu.get_barrier_semaphore()
    pl.semaphore_signal(barrier, inc=1, device_id=(left,),
                        device_id_type=pl.DeviceIdType.MESH)
    pl.semaphore_signal(barrier, inc=1, device_id=(right,),
                        device_id_type=pl.DeviceIdType.MESH)
    pl.semaphore_wait(barrier, 2)

    cp = pltpu.make_async_copy(x_ref, out_ref.at[my], local_sem)
    cp.start()

    cp.wait()
    a_r = pltpu.make_async_remote_copy(
        src_ref=out_ref.at[my], dst_ref=out_ref.at[my],
        send_sem=send_r0, recv_sem=recv_r0,
        device_id=(right,), device_id_type=pl.DeviceIdType.MESH,
    )
    a_l = pltpu.make_async_remote_copy(
        src_ref=out_ref.at[my], dst_ref=out_ref.at[my],
        send_sem=send_l0, recv_sem=recv_l0,
        device_id=(left,), device_id_type=pl.DeviceIdType.MESH,
    )
    a_r.start()
    a_l.start()

    b_r = pltpu.make_async_remote_copy(
        src_ref=out_ref.at[lm1, pl.ds(0, half)],
        dst_ref=out_ref.at[lm1, pl.ds(0, half)],
        send_sem=send_r1, recv_sem=recv_r1,
        device_id=(right,), device_id_type=pl.DeviceIdType.MESH,
    )
    b_l = pltpu.make_async_remote_copy(
        src_ref=out_ref.at[rp1, pl.ds(half, half)],
        dst_ref=out_ref.at[rp1, pl.ds(half, half)],
        send_sem=send_l1, recv_sem=recv_l1,
        device_id=(left,), device_id_type=pl.DeviceIdType.MESH,
    )
    a_r.wait_recv()
    b_r.start()
    a_l.wait_recv()
    b_l.start()
    b_r.wait_recv()
    b_l.wait_recv()

    a_r.wait_send()
    a_l.wait_send()
    b_r.wait_send()
    b_l.wait_send()


def kernel(x, w_mat):
    m_per, k = x.shape

    gathered = pl.pallas_call(
        _ag_body,
        out_shape=jax.ShapeDtypeStruct((N_DEV, m_per, k), x.dtype),
        in_specs=[pl.BlockSpec(memory_space=pltpu.MemorySpace.HBM)],
        out_specs=pl.BlockSpec(memory_space=pltpu.MemorySpace.HBM),
        scratch_shapes=[pltpu.SemaphoreType.DMA] * 9,
        compiler_params=pltpu.CompilerParams(collective_id=0),
    )(x)

    x_full = gathered.reshape(N_DEV * m_per, k)
    y = jnp.dot(x_full, w_mat, preferred_element_type=jnp.float32)
    return y * jax.nn.sigmoid(y)


# device time: 1339389 ns/iter; 3.2897x vs baseline; 2.4819x over previous
import jax
import jax.numpy as jnp
from jax import lax
from jax.experimental import pallas as pl
from jax.experimental.pallas import tpu as pltpu

N_DEV = 4


def _ag_body(x_ref, out_ref, send_r0, recv_r0, send_l0, recv_l0,
             send_r1, recv_r1, send_l1, recv_l1):
    m_per = x_ref.shape[0]
    half = m_per // 2
    my = lax.axis_index("i")
    left = lax.rem(my + (N_DEV - 1), N_DEV)
    right = lax.rem(my + 1, N_DEV)

    barrier = pltpu.get_barrier_semaphore()
    pl.semaphore_signal(barrier, inc=1, device_id=(left,),
                        device_id_type=pl.DeviceIdType.MESH)
    pl.semaphore_signal(barrier, inc=1, device_id=(right,),
                        device_id_type=pl.DeviceIdType.MESH)
    pl.semaphore_wait(barrier, 2)

    a_r = pltpu.make_async_remote_copy(
        src_ref=x_ref, dst_ref=out_ref.at[0],
        send_sem=send_r0, recv_sem=recv_r0,
        device_id=(right,), device_id_type=pl.DeviceIdType.MESH,
    )
    a_l = pltpu.make_async_remote_copy(
        src_ref=x_ref, dst_ref=out_ref.at[1],
        send_sem=send_l0, recv_sem=recv_l0,
        device_id=(left,), device_id_type=pl.DeviceIdType.MESH,
    )
    a_r.start()
    a_l.start()

    b_r = pltpu.make_async_remote_copy(
        src_ref=out_ref.at[0, pl.ds(0, half)],
        dst_ref=out_ref.at[2, pl.ds(0, half)],
        send_sem=send_r1, recv_sem=recv_r1,
        device_id=(right,), device_id_type=pl.DeviceIdType.MESH,
    )
    b_l = pltpu.make_async_remote_copy(
        src_ref=out_ref.at[1, pl.ds(half, half)],
        dst_ref=out_ref.at[2, pl.ds(half, half)],
        send_sem=send_l1, recv_sem=recv_l1,
        device_id=(left,), device_id_type=pl.DeviceIdType.MESH,
    )
    a_r.wait_recv()
    b_r.start()
    a_l.wait_recv()
    b_l.start()
    b_r.wait_recv()
    b_l.wait_recv()

    a_r.wait_send()
    a_l.wait_send()
    b_r.wait_send()
    b_l.wait_send()


def kernel(x, w_mat):
    m_per, k = x.shape

    gathered = pl.pallas_call(
        _ag_body,
        out_shape=jax.ShapeDtypeStruct((N_DEV - 1, m_per, k), x.dtype),
        in_specs=[pl.BlockSpec(memory_space=pltpu.MemorySpace.HBM)],
        out_specs=pl.BlockSpec(memory_space=pltpu.MemorySpace.HBM),
        scratch_shapes=[pltpu.SemaphoreType.DMA] * 8,
        compiler_params=pltpu.CompilerParams(collective_id=0),
    )(x)

    my = lax.axis_index("i")

    def block(xb):
        y = jnp.dot(xb, w_mat, preferred_element_type=jnp.float32)
        return y * jax.nn.sigmoid(y)

    y_rel = jnp.concatenate(
        [block(x), block(gathered[1]), block(gathered[2]), block(gathered[0])],
        axis=0,
    )
    return jnp.roll(y_rel, my * m_per, axis=0)


# device time: 985480 ns/iter; 4.4711x vs baseline; 1.3591x over previous
import jax
import jax.numpy as jnp
from jax import lax
from jax.experimental import pallas as pl
from jax.experimental.pallas import tpu as pltpu

N_DEV = 4


def _ag_w_body(w_ref, out_ref, send_r0, recv_r0, send_l0, recv_l0,
               send_r1, recv_r1, send_l1, recv_l1):
    k = w_ref.shape[0]
    half = k // 2
    my = lax.axis_index("i")
    left = lax.rem(my + (N_DEV - 1), N_DEV)
    right = lax.rem(my + 1, N_DEV)

    barrier = pltpu.get_barrier_semaphore()
    pl.semaphore_signal(barrier, inc=1, device_id=(left,),
                        device_id_type=pl.DeviceIdType.MESH)
    pl.semaphore_signal(barrier, inc=1, device_id=(right,),
                        device_id_type=pl.DeviceIdType.MESH)
    pl.semaphore_wait(barrier, 2)

    a_r = pltpu.make_async_remote_copy(
        src_ref=w_ref, dst_ref=out_ref.at[0],
        send_sem=send_r0, recv_sem=recv_r0,
        device_id=(right,), device_id_type=pl.DeviceIdType.MESH,
    )
    a_l = pltpu.make_async_remote_copy(
        src_ref=w_ref, dst_ref=out_ref.at[1],
        send_sem=send_l0, recv_sem=recv_l0,
        device_id=(left,), device_id_type=pl.DeviceIdType.MESH,
    )
    a_r.start()
    a_l.start()

    b_r = pltpu.make_async_remote_copy(
        src_ref=out_ref.at[0, pl.ds(0, half)],
        dst_ref=out_ref.at[2, pl.ds(0, half)],
        send_sem=send_r1, recv_sem=recv_r1,
        device_id=(right,), device_id_type=pl.DeviceIdType.MESH,
    )
    b_l = pltpu.make_async_remote_copy(
        src_ref=out_ref.at[1, pl.ds(half, half)],
        dst_ref=out_ref.at[2, pl.ds(half, half)],
        send_sem=send_l1, recv_sem=recv_l1,
        device_id=(left,), device_id_type=pl.DeviceIdType.MESH,
    )
    a_r.wait_recv()
    b_r.start()
    a_l.wait_recv()
    b_l.start()
    b_r.wait_recv()
    b_l.wait_recv()

    a_r.wait_send()
    a_l.wait_send()
    b_r.wait_send()
    b_l.wait_send()


def _a2a_body(y_l_ref, y_r_ref, y_d_ref, out_ref,
              send_l, recv_l, send_r, recv_r, send_d, recv_d):
    my = lax.axis_index("i")
    left = lax.rem(my + (N_DEV - 1), N_DEV)
    right = lax.rem(my + 1, N_DEV)
    diag = lax.rem(my + 2, N_DEV)

    barrier = pltpu.get_barrier_semaphore()
    for nbr in (left, right, diag):
        pl.semaphore_signal(barrier, inc=1, device_id=(nbr,),
                            device_id_type=pl.DeviceIdType.MESH)
    pl.semaphore_wait(barrier, 3)

    s_l = pltpu.make_async_remote_copy(
        src_ref=y_l_ref, dst_ref=out_ref.at[1],
        send_sem=send_l, recv_sem=recv_l,
        device_id=(left,), device_id_type=pl.DeviceIdType.MESH,
    )
    s_r = pltpu.make_async_remote_copy(
        src_ref=y_r_ref, dst_ref=out_ref.at[0],
        send_sem=send_r, recv_sem=recv_r,
        device_id=(right,), device_id_type=pl.DeviceIdType.MESH,
    )
    s_d = pltpu.make_async_remote_copy(
        src_ref=y_d_ref, dst_ref=out_ref.at[2],
        send_sem=send_d, recv_sem=recv_d,
        device_id=(diag,), device_id_type=pl.DeviceIdType.MESH,
    )
    s_l.start()
    s_r.start()
    s_d.start()
    s_l.wait_recv()
    s_r.wait_recv()
    s_d.wait_recv()
    s_l.wait_send()
    s_r.wait_send()
    s_d.wait_send()


def kernel(x, w_mat):
    m_per, k = x.shape
    n_per = w_mat.shape[1]
    my = lax.axis_index("i")

    w_gathered = pl.pallas_call(
        _ag_w_body,
        out_shape=jax.ShapeDtypeStruct((N_DEV - 1, k, n_per), w_mat.dtype),
        in_specs=[pl.BlockSpec(memory_space=pltpu.MemorySpace.HBM)],
        out_specs=pl.BlockSpec(memory_space=pltpu.MemorySpace.HBM),
        scratch_shapes=[pltpu.SemaphoreType.DMA] * 8,
        compiler_params=pltpu.CompilerParams(collective_id=0),
    )(w_mat)

    def block(wb):
        y = jnp.dot(x, wb, preferred_element_type=jnp.float32)
        return y * jax.nn.sigmoid(y)

    y_own = block(w_mat)
    y_l = block(w_gathered[0])
    y_r = block(w_gathered[1])
    y_d = block(w_gathered[2])

    y_recv = pl.pallas_call(
        _a2a_body,
        out_shape=jax.ShapeDtypeStruct((N_DEV - 1, m_per, n_per), jnp.float32),
        in_specs=[pl.BlockSpec(memory_space=pltpu.MemorySpace.HBM)] * 3,
        out_specs=pl.BlockSpec(memory_space=pltpu.MemorySpace.HBM),
        scratch_shapes=[pltpu.SemaphoreType.DMA] * 6,
        compiler_params=pltpu.CompilerParams(collective_id=1),
    )(y_l, y_r, y_d)

    y_rel = jnp.concatenate(
        [y_own, y_recv[1], y_recv[2], y_recv[0]], axis=0)
    return jnp.roll(y_rel, my * m_per, axis=0)


# device time: 751414 ns/iter; 5.8638x vs baseline; 1.3115x over previous
import jax
import jax.numpy as jnp
from jax import lax
from jax.experimental import pallas as pl
from jax.experimental.pallas import tpu as pltpu

N_DEV = 4
KT = 1024


def _gemm_stream(x_ref, w_hbm, acc_ref, xt_ref, wt_ref, xt_sems, wt_sems):
    k = x_ref.shape[1]
    nkt = k // KT

    def _tiles(t, buf):
        cx = pltpu.make_async_copy(
            x_ref.at[:, pl.ds(t * KT, KT)], xt_ref.at[buf], xt_sems.at[buf])
        cw = pltpu.make_async_copy(
            w_hbm.at[pl.ds(t * KT, KT), :], wt_ref.at[buf], wt_sems.at[buf])
        return cx, cw

    def _issue(t, buf):
        cx, cw = _tiles(t, buf)
        cx.start()
        cw.start()

    _issue(0, 0)
    _issue(1, 1)
    acc_ref[...] = jnp.zeros_like(acc_ref)

    def _pair(i, carry):
        t0 = i * 2
        for off, buf in ((0, 0), (1, 1)):
            t = t0 + off
            cx, cw = _tiles(t, buf)
            cx.wait()
            cw.wait()

            acc_ref[...] += jnp.dot(xt_ref[buf], wt_ref[buf],
                                    preferred_element_type=jnp.float32)

            nxt = t + 2

            @pl.when(nxt < nkt)
            def _():
                _issue(nxt, buf)
        return carry

    lax.fori_loop(0, nkt // 2, _pair, 0)
    y = acc_ref[...]
    acc_ref[...] = y * (1.0 / (1.0 + jnp.exp(-y)))


def _body(x_ref, w_ref, out_ref, wslots_ref, yd_hbm_ref,
          acc_own, acc_l, acc_r, acc_d, xt_ref, wt_ref,
          xt_sems, wt_sems, own_sem, yd_sem,
          send_r0, recv_r0, send_l0, recv_l0,
          send_r1, recv_r1, send_l1, recv_l1,
          send_yl, recv_yl, send_yr, recv_yr, send_yd, recv_yd):
    m_per = x_ref.shape[0]
    k = x_ref.shape[1]
    half = k // 2
    my = lax.axis_index("i")
    left = lax.rem(my + (N_DEV - 1), N_DEV)
    right = lax.rem(my + 1, N_DEV)
    diag = lax.rem(my + 2, N_DEV)

    barrier = pltpu.get_barrier_semaphore()
    for nbr in (left, right, diag):
        pl.semaphore_signal(barrier, inc=1, device_id=(nbr,),
                            device_id_type=pl.DeviceIdType.MESH)
    pl.semaphore_wait(barrier, 3)

    a_r = pltpu.make_async_remote_copy(
        src_ref=w_ref, dst_ref=wslots_ref.at[0],
        send_sem=send_r0, recv_sem=recv_r0,
        device_id=(right,), device_id_type=pl.DeviceIdType.MESH,
    )
    a_l = pltpu.make_async_remote_copy(
        src_ref=w_ref, dst_ref=wslots_ref.at[1],
        send_sem=send_l0, recv_sem=recv_l0,
        device_id=(left,), device_id_type=pl.DeviceIdType.MESH,
    )
    a_r.start()
    a_l.start()

    _gemm_stream(x_ref, w_ref, acc_own, xt_ref, wt_ref, xt_sems, wt_sems)
    dma_own = pltpu.make_async_copy(
        acc_own, out_ref.at[pl.ds(my * m_per, m_per)], own_sem)
    dma_own.start()

    b_r = pltpu.make_async_remote_copy(
        src_ref=wslots_ref.at[0, pl.ds(0, half)],
        dst_ref=wslots_ref.at[2, pl.ds(0, half)],
        send_sem=send_r1, recv_sem=recv_r1,
        device_id=(right,), device_id_type=pl.DeviceIdType.MESH,
    )
    b_l = pltpu.make_async_remote_copy(
        src_ref=wslots_ref.at[1, pl.ds(half, half)],
        dst_ref=wslots_ref.at[2, pl.ds(half, half)],
        send_sem=send_l1, recv_sem=recv_l1,
        device_id=(left,), device_id_type=pl.DeviceIdType.MESH,
    )

    s_l = pltpu.make_async_remote_copy(
        src_ref=acc_l, dst_ref=out_ref.at[pl.ds(my * m_per, m_per)],
        send_sem=send_yl, recv_sem=recv_yl,
        device_id=(left,), device_id_type=pl.DeviceIdType.MESH,
    )
    s_r = pltpu.make_async_remote_copy(
        src_ref=acc_r, dst_ref=out_ref.at[pl.ds(my * m_per, m_per)],
        send_sem=send_yr, recv_sem=recv_yr,
        device_id=(right,), device_id_type=pl.DeviceIdType.MESH,
    )
    s_d = pltpu.make_async_remote_copy(
        src_ref=yd_hbm_ref, dst_ref=out_ref.at[pl.ds(my * m_per, m_per)],
        send_sem=send_yd, recv_sem=recv_yd,
        device_id=(diag,), device_id_type=pl.DeviceIdType.MESH,
    )

    a_r.wait_recv()
    b_r.start()
    a_l.wait_recv()
    b_l.start()

    _gemm_stream(x_ref, wslots_ref.at[0], acc_l, xt_ref, wt_ref,
                 xt_sems, wt_sems)
    s_l.start()
    _gemm_stream(x_ref, wslots_ref.at[1], acc_r, xt_ref, wt_ref,
                 xt_sems, wt_sems)
    s_r.start()

    b_r.wait_recv()
    b_l.wait_recv()
    _gemm_stream(x_ref, wslots_ref.at[2], acc_d, xt_ref, wt_ref,
                 xt_sems, wt_sems)
    stage_d = pltpu.make_async_copy(acc_d, yd_hbm_ref, yd_sem)
    stage_d.start()
    stage_d.wait()
    s_d.start()

    dma_own.wait()
    s_l.wait_recv()
    s_r.wait_recv()
    s_d.wait_recv()
    a_r.wait_send()
    a_l.wait_send()
    b_r.wait_send()
    b_l.wait_send()
    s_l.wait_send()
    s_r.wait_send()
    s_d.wait_send()


def kernel(x, w_mat):
    m_per, k = x.shape
    n_per = w_mat.shape[1]

    out, _, _ = pl.pallas_call(
        _body,
        out_shape=[
            jax.ShapeDtypeStruct((N_DEV * m_per, n_per), jnp.float32),
            jax.ShapeDtypeStruct((N_DEV - 1, k, n_per), w_mat.dtype),
            jax.ShapeDtypeStruct((m_per, n_per), jnp.float32),
        ],
        in_specs=[pl.BlockSpec(memory_space=pltpu.MemorySpace.HBM)] * 2,
        out_specs=[pl.BlockSpec(memory_space=pltpu.MemorySpace.HBM)] * 3,
        scratch_shapes=(
            [pltpu.VMEM((m_per, n_per), jnp.float32)] * 4
            + [
                pltpu.VMEM((2, m_per, KT), jnp.float32),
                pltpu.VMEM((2, KT, n_per), jnp.float32),
                pltpu.SemaphoreType.DMA((2,)),
                pltpu.SemaphoreType.DMA((2,)),
                pltpu.SemaphoreType.DMA,
                pltpu.SemaphoreType.DMA,
            ]
            + [pltpu.SemaphoreType.DMA] * 14
        ),
        compiler_params=pltpu.CompilerParams(
            collective_id=0, vmem_limit_bytes=100 * 1024 * 1024),
    )(x, w_mat)
    return out
